# baseline (device time: 172295 ns/iter reference)
import jax
import jax.numpy as jnp
from jax import lax
from jax.experimental import pallas as pl
from jax.experimental.pallas import tpu as pltpu

N_DEV = 4
SUB = 8


def kernel(x):
    m_per, n = x.shape
    hm = m_per // 2
    sub_m = m_per // SUB

    def body(x_ref, out_ref, stage_ref, conv_ref, load_sems,
             sendr_sems, sendl_sems, recvl_sems, recvr_sems,
             p2_send_sems, p2_recv_sems, own_sem):
        my_pos = lax.axis_index("i")
        left = (my_pos - 1) % N_DEV
        right = (my_pos + 1) % N_DEV

        def load(k, slot):
            return pltpu.make_async_copy(
                x_ref.at[pl.ds(k * sub_m, sub_m)],
                stage_ref.at[slot],
                load_sems.at[slot],
            )

        load(0, 0).start()
        sends_r = []
        sends_l = []
        for k in range(SUB):
            load(k, k % 2).wait()
            if k + 1 < SUB:
                load(k + 1, (k + 1) % 2).start()
            conv_ref[pl.ds(k * sub_m, sub_m), :] = (
                stage_ref[k % 2, :, :].astype(jnp.bfloat16)
            )
            dst = out_ref.at[pl.ds(my_pos * m_per + k * sub_m, sub_m)]
            r = pltpu.make_async_remote_copy(
                src_ref=conv_ref.at[pl.ds(k * sub_m, sub_m)],
                dst_ref=dst,
                send_sem=sendr_sems.at[k],
                recv_sem=recvl_sems.at[k],
                device_id=(right,),
                device_id_type=pl.DeviceIdType.MESH,
            )
            l = pltpu.make_async_remote_copy(
                src_ref=conv_ref.at[pl.ds(k * sub_m, sub_m)],
                dst_ref=dst,
                send_sem=sendl_sems.at[k],
                recv_sem=recvr_sems.at[k],
                device_id=(left,),
                device_id_type=pl.DeviceIdType.MESH,
            )
            r.start()
            l.start()
            sends_r.append(r)
            sends_l.append(l)

        own_copy = pltpu.make_async_copy(
            conv_ref, out_ref.at[pl.ds(my_pos * m_per, m_per)], own_sem
        )
        own_copy.start()

        for k in range(SUB // 2):
            sends_r[k].wait_recv()
        p2r = pltpu.make_async_remote_copy(
            src_ref=out_ref.at[pl.ds(left * m_per, hm)],
            dst_ref=out_ref.at[pl.ds(left * m_per, hm)],
            send_sem=p2_send_sems.at[0],
            recv_sem=p2_recv_sems.at[0],
            device_id=(right,),
            device_id_type=pl.DeviceIdType.MESH,
        )
        p2r.start()
        for k in range(SUB // 2, SUB):
            sends_r[k].wait_recv()
        for k in range(SUB):
            sends_l[k].wait_recv()

        p2l = pltpu.make_async_remote_copy(
            src_ref=out_ref.at[pl.ds(right * m_per + hm, hm)],
            dst_ref=out_ref.at[pl.ds(right * m_per + hm, hm)],
            send_sem=p2_send_sems.at[1],
            recv_sem=p2_recv_sems.at[1],
            device_id=(left,),
            device_id_type=pl.DeviceIdType.MESH,
        )
        p2l.start()

        p2r.wait()
        p2l.wait()
        own_copy.wait()
        for k in range(SUB):
            sends_r[k].wait_send()
            sends_l[k].wait_send()

    return pl.pallas_call(
        body,
        out_shape=jax.ShapeDtypeStruct((N_DEV * m_per, n), jnp.bfloat16),
        in_specs=[pl.BlockSpec(memory_space=pl.ANY)],
        out_specs=pl.BlockSpec(memory_space=pl.ANY),
        scratch_shapes=[
            pltpu.VMEM((2, sub_m, n), jnp.float32),
            pltpu.VMEM((m_per, n), jnp.bfloat16),
            pltpu.SemaphoreType.DMA((2,)),
            pltpu.SemaphoreType.DMA((SUB,)),
            pltpu.SemaphoreType.DMA((SUB,)),
            pltpu.SemaphoreType.DMA((SUB,)),
            pltpu.SemaphoreType.DMA((SUB,)),
            pltpu.SemaphoreType.DMA((2,)),
            pltpu.SemaphoreType.DMA((2,)),
            pltpu.SemaphoreType.DMA,
        ],
    )(x)
